# baseline (speedup 1.0000x reference)
import functools

import jax
import jax.numpy as jnp
from jax.experimental import pallas as pl
from jax.experimental.pallas import tpu as pltpu


_GAMMA = 0.1


def _monotone_key(x):
    bits = jax.lax.bitcast_convert_type(x, jnp.int32)
    return bits ^ (jax.lax.shift_right_arithmetic(bits, 31) & jnp.int32(0x7FFFFFFF))


def _kwta_kernel(x_ref, o_ref, hi_ref, lo_ref, *, k):
    key = _monotone_key(x_ref[...])
    hi_ref[...] = jax.lax.shift_right_arithmetic(key, 16).astype(jnp.int16)
    lo_ref[...] = ((key & jnp.int32(0xFFFF)) - jnp.int32(32768)).astype(jnp.int16)

    rows, h, w = x_ref.shape
    hstep = 16

    def _count(m):
        if h % hstep:
            return jnp.sum(
                m.astype(jnp.int32), axis=(1, 2), keepdims=True
            )
        acc = functools.reduce(
            jnp.add,
            [m[:, j * hstep:(j + 1) * hstep, :] for j in range(h // hstep)],
        )
        return jnp.sum(acc.astype(jnp.int32), axis=(1, 2), keepdims=True)

    half = rows // 2

    def _descend(body_count, k_eff):
        def body(b, carry):
            ta, tb = carry
            bit = jnp.left_shift(jnp.int32(1), jnp.int32(15) - b)
            ca = ta + bit
            cb = tb + bit
            cnt_a = body_count(0, half, ca.astype(jnp.int16))
            cnt_b = body_count(half, rows, cb.astype(jnp.int16))
            return (
                jnp.where(cnt_a >= k_eff, ca, ta),
                jnp.where(cnt_b >= k_eff, cb, tb),
            )

        t0 = jnp.full((half, 1, 1), jnp.int32(-32768), jnp.int32)
        ta, tb = jax.lax.fori_loop(0, 16, body, (t0, t0))
        return jnp.concatenate([ta, tb], axis=0)

    p = _descend(
        lambda r0, r1, cand: _count(
            (hi_ref[r0:r1] >= cand).astype(jnp.int16)
        ),
        k,
    )
    p16 = p.astype(jnp.int16)

    a_cnt = _count((hi_ref[...] > p16).astype(jnp.int16))

    lo_ref[...] = jnp.where(hi_ref[...] == p16, lo_ref[...], jnp.int16(-32768))

    t2 = _descend(
        lambda r0, r1, cand: a_cnt[r0:r1]
        + _count((lo_ref[r0:r1] >= cand).astype(jnp.int16)),
        k,
    )

    t_full = jnp.left_shift(p, 16) | ((t2 ^ jnp.int32(0x8000)) & jnp.int32(0xFFFF))

    x = x_ref[...]
    key2 = _monotone_key(x)
    o_ref[...] = jnp.where(key2 >= t_full, jnp.float32(0.0), x)


@jax.jit
def kernel(x):
    B, C, H, W = x.shape
    n = H * W
    k = int(_GAMMA * n)
    rows_total = B * C
    block_rows = 8
    assert rows_total % block_rows == 0
    x3 = x.reshape(rows_total, H, W)
    out = pl.pallas_call(
        functools.partial(_kwta_kernel, k=k),
        grid=(rows_total // block_rows,),
        in_specs=[pl.BlockSpec((block_rows, H, W), lambda i: (i, 0, 0))],
        out_specs=pl.BlockSpec((block_rows, H, W), lambda i: (i, 0, 0)),
        out_shape=jax.ShapeDtypeStruct((rows_total, H, W), x.dtype),
        scratch_shapes=[
            pltpu.VMEM((block_rows, H, W), jnp.int16),
            pltpu.VMEM((block_rows, H, W), jnp.int16),
        ],
    )(x3)
    return out.reshape(B, C, H, W)


# speedup vs baseline: 1.2563x; 1.2563x over previous
import functools

import jax
import jax.numpy as jnp
from jax.experimental import pallas as pl
from jax.experimental.pallas import tpu as pltpu


_GAMMA = 0.1


def _monotone_key(x):
    bits = jax.lax.bitcast_convert_type(x, jnp.int32)
    return bits ^ (jax.lax.shift_right_arithmetic(bits, 31) & jnp.int32(0x7FFFFFFF))


def _kwta_kernel(x_ref, o_ref, hi_ref, lo_ref, *, k):
    key = _monotone_key(x_ref[...])
    hi_ref[...] = jax.lax.shift_right_arithmetic(key, 16).astype(jnp.int16)
    lo_ref[...] = ((key & jnp.int32(0xFFFF)) - jnp.int32(32768)).astype(jnp.int16)

    rows, h, w = x_ref.shape
    hstep = 16

    def _count(m):
        if h % hstep:
            return jnp.sum(
                m.astype(jnp.int32), axis=(1, 2), keepdims=True
            )
        acc = functools.reduce(
            jnp.add,
            [m[:, j * hstep:(j + 1) * hstep, :] for j in range(h // hstep)],
        )
        return jnp.sum(acc.astype(jnp.int32), axis=(1, 2), keepdims=True)

    def _descend(body_count, k_eff):
        def body(b, t):
            bit = jnp.left_shift(jnp.int32(1), jnp.int32(15) - b)
            cand = t + bit
            cnt = body_count(cand.astype(jnp.int16))
            return jnp.where(cnt >= k_eff, cand, t)

        t0 = jnp.full((rows, 1, 1), jnp.int32(-32768), jnp.int32)
        return jax.lax.fori_loop(0, 16, body, t0)

    p = _descend(lambda cand: _count((hi_ref[...] >= cand).astype(jnp.int16)), k)
    p16 = p.astype(jnp.int16)

    hi = hi_ref[...]
    lo_ref[...] = jnp.where(
        hi == p16,
        lo_ref[...],
        jnp.where(hi > p16, jnp.int16(32767), jnp.int16(-32768)),
    )

    t2 = _descend(lambda cand: _count((lo_ref[...] >= cand).astype(jnp.int16)), k)

    t_full = jnp.left_shift(p, 16) | ((t2 ^ jnp.int32(0x8000)) & jnp.int32(0xFFFF))
    t_bits = t_full ^ (jax.lax.shift_right_arithmetic(t_full, 31) & jnp.int32(0x7FFFFFFF))
    thresh = jax.lax.bitcast_convert_type(t_bits, jnp.float32)

    x = x_ref[...]
    o_ref[...] = jnp.where(x >= thresh, jnp.float32(0.0), x)


@jax.jit
def kernel(x):
    B, C, H, W = x.shape
    n = H * W
    k = int(_GAMMA * n)
    rows_total = B * C
    block_rows = 16
    assert rows_total % block_rows == 0
    x3 = x.reshape(rows_total, H, W)
    out = pl.pallas_call(
        functools.partial(_kwta_kernel, k=k),
        grid=(rows_total // block_rows,),
        in_specs=[pl.BlockSpec((block_rows, H, W), lambda i: (i, 0, 0))],
        out_specs=pl.BlockSpec((block_rows, H, W), lambda i: (i, 0, 0)),
        out_shape=jax.ShapeDtypeStruct((rows_total, H, W), x.dtype),
        scratch_shapes=[
            pltpu.VMEM((block_rows, H, W), jnp.int16),
            pltpu.VMEM((block_rows, H, W), jnp.int16),
        ],
    )(x3)
    return out.reshape(B, C, H, W)
